# Initial kernel scaffold; baseline (speedup 1.0000x reference)
import jax
import jax.numpy as jnp
from jax.experimental import pallas as pl


def kernel(feats0, feats1, pos_pairs):
    raise NotImplementedError("write your pallas kernel here")


# speedup vs baseline: 5.7093x; 5.7093x over previous
import functools

import jax
import jax.numpy as jnp
import numpy as np
from jax.experimental import pallas as pl
from jax.experimental.pallas import tpu as pltpu

POS_THRESH = 0.1
NEG_THRESH = 20.0
NPOS = 4096
NCAND = 16384
NPAIRS = 10000
NROWS = 100000
D = 128

_BM = 512
_BN = 2048
_CH = 256
_PPAD = 10240
_I0 = np.int32(0)

_INTERPRET = False


def _threefry2x32(k1, k2, c1, c2):
    def rotl(x, d):
        return ((x << np.uint32(d)) | (x >> np.uint32(32 - d))).astype(
            np.uint32)

    rot_a = (13, 15, 26, 6)
    rot_b = (17, 29, 16, 24)
    ks0, ks1 = np.uint32(k1), np.uint32(k2)
    ks2 = np.uint32(ks0 ^ ks1 ^ np.uint32(0x1BD11BDA))
    x0 = (c1 + ks0).astype(np.uint32)
    x1 = (c2 + ks1).astype(np.uint32)

    def rounds(x0, x1, rs):
        for r in rs:
            x0 = (x0 + x1).astype(np.uint32)
            x1 = x0 ^ rotl(x1, r)
        return x0, x1

    for i, (ka_, kb_, rs) in enumerate(
            ((ks1, ks2, rot_a), (ks2, ks0, rot_b), (ks0, ks1, rot_a),
             (ks1, ks2, rot_b), (ks2, ks0, rot_a))):
        x0, x1 = rounds(x0, x1, rs)
        x0 = (x0 + ka_).astype(np.uint32)
        x1 = (x1 + kb_ + np.uint32(i + 1)).astype(np.uint32)
    return x0, x1


def _np_split(key, num):
    b1, b2 = _threefry2x32(key[0], key[1], np.zeros(num, np.uint32),
                           np.arange(num, dtype=np.uint32))
    return np.stack([b1, b2], axis=1)


def _np_choice_noreplace(key, n, k):
    x = np.arange(n, dtype=np.int32)
    num_rounds = int(np.ceil(3 * np.log(max(1, n)) / np.log(0xFFFFFFFF)))
    for _ in range(num_rounds):
        key, subkey = _np_split(key, 2)
        b1, b2 = _threefry2x32(subkey[0], subkey[1], np.zeros(n, np.uint32),
                               np.arange(n, dtype=np.uint32))
        x = x[np.argsort(b1 ^ b2, kind="stable")]
    return x[:k]


def _sampling_constants():
    ka, kb, kc = _np_split(np.array([0, 42], np.uint32), 3)
    return (_np_choice_noreplace(ka, NPAIRS, NPOS),
            _np_choice_noreplace(kb, NROWS, NCAND),
            _np_choice_noreplace(kc, NROWS, NCAND))


_SEL, _CAND0, _CAND1 = _sampling_constants()


def _dist_body(a_ref, b_ref, o_ref, mind_ref, mini_ref):
    a = a_ref[0]
    pa = jnp.sum(a * a, axis=1, keepdims=True)

    def step(j, carry):
        mind, mini = carry
        b = b_ref[0, pl.ds(j * _BN, _BN), :]
        g = jax.lax.dot_general(a, b, (((1,), (1,)), ((), ())),
                                preferred_element_type=jnp.float32)
        cb = jnp.sum(b * b, axis=1)[None, :]
        raw = pa + cb - 2.0 * g
        tmin = jnp.min(raw, axis=1, keepdims=True)
        o = o_ref[0, 0, pl.ds(j * _BN, _BN)][None, :]
        tidx = jnp.min(jnp.where(raw == tmin, o, jnp.inf), axis=1,
                       keepdims=True)
        upd = tmin < mind
        return jnp.where(upd, tmin, mind), jnp.where(upd, tidx, mini)

    mind, mini = jax.lax.fori_loop(
        jnp.int32(0), jnp.int32(NCAND // _BN), step,
        (jnp.full((_BM, 1), jnp.inf, jnp.float32),
         jnp.zeros((_BM, 1), jnp.float32)))
    mind_ref[0] = mind
    mini_ref[0] = mini


def _finalize_body(p0_ref, p1_ref, posidx_ref, mini_ref, mind_ref,
                   pos0_ref, pos1_ref, tot_ref, pos_ref, neg_ref):
    p0 = p0_ref[...]
    p1 = p1_ref[...]

    def chunk(i, carry):
        s0, c0, s1, c1 = carry
        sl = pl.ds(i * _CH, _CH)
        u0 = posidx_ref[0, sl, :]
        v0 = mini_ref[0, sl, :]
        nhit0 = jnp.sum(((u0 == p0) & (v0 == p1)).astype(jnp.float32),
                        axis=1, keepdims=True)
        d0 = jnp.sqrt(jnp.maximum(mind_ref[0, sl, :], 1e-12))
        t0 = jnp.maximum(NEG_THRESH - d0, 0.0) ** 2
        w0 = (nhit0 == 0).astype(jnp.float32)
        u1 = mini_ref[1, sl, :]
        v1 = posidx_ref[1, sl, :]
        nhit1 = jnp.sum(((u1 == p0) & (v1 == p1)).astype(jnp.float32),
                        axis=1, keepdims=True)
        d1 = jnp.sqrt(jnp.maximum(mind_ref[1, sl, :], 1e-12))
        t1 = jnp.maximum(NEG_THRESH - d1, 0.0) ** 2
        w1 = (nhit1 == 0).astype(jnp.float32)
        return (s0 + jnp.sum(t0 * w0), c0 + jnp.sum(w0),
                s1 + jnp.sum(t1 * w1), c1 + jnp.sum(w1))

    z = jnp.float32(0.0)
    s0, c0, s1, c1 = jax.lax.fori_loop(jnp.int32(0), jnp.int32(NPOS // _CH),
                                       chunk, (z, z, z, z))

    diff = pos0_ref[...] - pos1_ref[...]
    nrm = jnp.sqrt(jnp.sum(diff * diff, axis=1))
    ps = jnp.sum(jnp.maximum(nrm - POS_THRESH, 0.0) ** 2) / NPOS

    nl = (s0 / jnp.maximum(c0, 1.0) + s1 / jnp.maximum(c1, 1.0)) / 2.0
    pos_ref[...] = ps.reshape(1, 1)
    neg_ref[...] = nl.reshape(1, 1)
    tot_ref[...] = (ps + nl).reshape(1, 1)


def kernel(feats0, feats1, pos_pairs):
    pp = pos_pairs.astype(jnp.int32)
    pos_idx0 = pp[_SEL, 0]
    pos_idx1 = pp[_SEL, 1]

    A = jnp.stack([feats0[pos_idx0], feats1[pos_idx1]])
    B = jnp.stack([feats1[jnp.asarray(_CAND1)],
                   feats0[jnp.asarray(_CAND0)]])

    orig = jnp.stack([jnp.asarray(_CAND1, jnp.int32),
                      jnp.asarray(_CAND0, jnp.int32)]).astype(jnp.float32)
    orig = orig[:, None, :]

    grid = (2, NPOS // _BM)
    mind, mini = pl.pallas_call(
        _dist_body,
        grid=grid,
        in_specs=[
            pl.BlockSpec((1, _BM, D), lambda d, m: (d, m, _I0)),
            pl.BlockSpec((1, NCAND, D), lambda d, m: (d, _I0, _I0)),
            pl.BlockSpec((1, 1, NCAND), lambda d, m: (d, _I0, _I0)),
        ],
        out_specs=[
            pl.BlockSpec((1, _BM, 1), lambda d, m: (d, m, _I0)),
            pl.BlockSpec((1, _BM, 1), lambda d, m: (d, m, _I0)),
        ],
        out_shape=[
            jax.ShapeDtypeStruct((2, NPOS, 1), jnp.float32),
            jax.ShapeDtypeStruct((2, NPOS, 1), jnp.float32),
        ],
        interpret=_INTERPRET,
    )(A, B, orig)

    pad = jnp.full((_PPAD - NPAIRS,), -1.0, jnp.float32)
    p0f = jnp.concatenate([pp[:, 0].astype(jnp.float32), pad])[None, :]
    p1f = jnp.concatenate([pp[:, 1].astype(jnp.float32), pad])[None, :]
    posidx = jnp.stack([pos_idx0, pos_idx1]).astype(jnp.float32)[:, :, None]

    tot, posl, negl = pl.pallas_call(
        _finalize_body,
        out_shape=[
            jax.ShapeDtypeStruct((1, 1), jnp.float32),
            jax.ShapeDtypeStruct((1, 1), jnp.float32),
            jax.ShapeDtypeStruct((1, 1), jnp.float32),
        ],
        interpret=_INTERPRET,
    )(p0f, p1f, posidx, mini, mind, A[0], A[1])

    return tot[0, 0], posl[0, 0], negl[0, 0]


# speedup vs baseline: 6.2476x; 1.0943x over previous
import functools

import jax
import jax.numpy as jnp
import numpy as np
from jax import lax
from jax.experimental import pallas as pl
from jax.experimental.pallas import tpu as pltpu
from jax.experimental.pallas import tpu_sc as plsc

POS_THRESH = 0.1
NEG_THRESH = 20.0
NPOS = 4096
NCAND = 16384
NPAIRS = 10000
NROWS = 100000
D = 128

_BM = 512
_BN = 2048
_CH = 256
_PPAD = 10240
_I0 = np.int32(0)

_INTERPRET = False


def _threefry2x32(k1, k2, c1, c2):
    def rotl(x, d):
        return ((x << np.uint32(d)) | (x >> np.uint32(32 - d))).astype(
            np.uint32)

    rot_a = (13, 15, 26, 6)
    rot_b = (17, 29, 16, 24)
    ks0, ks1 = np.uint32(k1), np.uint32(k2)
    ks2 = np.uint32(ks0 ^ ks1 ^ np.uint32(0x1BD11BDA))
    x0 = (c1 + ks0).astype(np.uint32)
    x1 = (c2 + ks1).astype(np.uint32)

    def rounds(x0, x1, rs):
        for r in rs:
            x0 = (x0 + x1).astype(np.uint32)
            x1 = x0 ^ rotl(x1, r)
        return x0, x1

    for i, (ka_, kb_, rs) in enumerate(
            ((ks1, ks2, rot_a), (ks2, ks0, rot_b), (ks0, ks1, rot_a),
             (ks1, ks2, rot_b), (ks2, ks0, rot_a))):
        x0, x1 = rounds(x0, x1, rs)
        x0 = (x0 + ka_).astype(np.uint32)
        x1 = (x1 + kb_ + np.uint32(i + 1)).astype(np.uint32)
    return x0, x1


def _np_split(key, num):
    b1, b2 = _threefry2x32(key[0], key[1], np.zeros(num, np.uint32),
                           np.arange(num, dtype=np.uint32))
    return np.stack([b1, b2], axis=1)


def _np_choice_noreplace(key, n, k):
    x = np.arange(n, dtype=np.int32)
    num_rounds = int(np.ceil(3 * np.log(max(1, n)) / np.log(0xFFFFFFFF)))
    for _ in range(num_rounds):
        key, subkey = _np_split(key, 2)
        b1, b2 = _threefry2x32(subkey[0], subkey[1], np.zeros(n, np.uint32),
                               np.arange(n, dtype=np.uint32))
        x = x[np.argsort(b1 ^ b2, kind="stable")]
    return x[:k]


def _sampling_constants():
    ka, kb, kc = _np_split(np.array([0, 42], np.uint32), 3)
    return (_np_choice_noreplace(ka, NPAIRS, NPOS),
            _np_choice_noreplace(kb, NROWS, NCAND),
            _np_choice_noreplace(kc, NROWS, NCAND))


_SEL, _CAND0, _CAND1 = _sampling_constants()


_NW = 32
_GC = 128
_NBUF = 4


def _sc_gather_body(f0_ref, f1_ref, idx_ref, posa_ref, candb_ref,
                    idx_v, bufs, gsem, psem):
    w = (lax.axis_index("s") * 2 + lax.axis_index("c")).astype(jnp.int32)

    i32 = np.int32
    pltpu.sync_copy(idx_ref.at[w], idx_v.at[i32(0)])
    pltpu.sync_copy(idx_ref.at[i32(32) + w], idx_v.at[i32(1)])
    pltpu.sync_copy(idx_ref.at[pl.ds(i32(64) + i32(4) * w, 4)],
                    idx_v.at[pl.ds(i32(2), 4)])
    pltpu.sync_copy(idx_ref.at[pl.ds(i32(192) + i32(4) * w, 4)],
                    idx_v.at[pl.ds(i32(6), 4)])

    chunks = []
    chunks.append((f0_ref, 0, posa_ref.at[i32(0), pl.ds(w * i32(_GC), _GC)]))
    chunks.append((f1_ref, 1, posa_ref.at[i32(1), pl.ds(w * i32(_GC), _GC)]))
    for t in range(4):
        chunks.append((f1_ref, 2 + t,
                       candb_ref.at[i32(0),
                                    pl.ds(w * i32(4 * _GC) + i32(t * _GC),
                                          _GC)]))
    for t in range(4):
        chunks.append((f0_ref, 6 + t,
                       candb_ref.at[i32(1),
                                    pl.ds(w * i32(4 * _GC) + i32(t * _GC),
                                          _GC)]))

    n = len(chunks)
    gathers = [None] * n
    puts = [None] * n
    for i in range(min(_NBUF, n)):
        tbl, r, _ = chunks[i]
        gathers[i] = pltpu.async_copy(tbl.at[idx_v.at[np.int32(r)]],
                                      bufs[i % _NBUF], gsem[i % _NBUF])
    for i in range(n):
        gathers[i].wait()
        _, _, dst = chunks[i]
        puts[i] = pltpu.async_copy(bufs[i % _NBUF], dst, psem[i % _NBUF])
        j = i + _NBUF
        if j < n:
            puts[i].wait()
            tbl, r, _ = chunks[j]
            gathers[j] = pltpu.async_copy(tbl.at[idx_v.at[np.int32(r)]],
                                          bufs[j % _NBUF], gsem[j % _NBUF])
    for i in range(max(0, n - _NBUF), n):
        puts[i].wait()


def _sc_gather(feats0, feats1, idx_all):
    mesh = plsc.VectorSubcoreMesh(core_axis_name="c", subcore_axis_name="s")
    return pl.kernel(
        _sc_gather_body,
        out_type=[
            jax.ShapeDtypeStruct((2, NPOS, D), jnp.float32),
            jax.ShapeDtypeStruct((2, NCAND, D), jnp.float32),
        ],
        mesh=mesh,
        scratch_types=[
            pltpu.VMEM((10, _GC), jnp.int32),
            [pltpu.VMEM((_GC, D), jnp.float32) for _ in range(_NBUF)],
            [pltpu.SemaphoreType.DMA for _ in range(_NBUF)],
            [pltpu.SemaphoreType.DMA for _ in range(_NBUF)],
        ],
    )(feats0, feats1, idx_all)


def _dist_body(a_ref, b_ref, o_ref, mind_ref, mini_ref):
    a = a_ref[0]
    pa = jnp.sum(a * a, axis=1, keepdims=True)

    def step(j, carry):
        mind, mini = carry
        b = b_ref[0, pl.ds(j * _BN, _BN), :]
        g = jax.lax.dot_general(a, b, (((1,), (1,)), ((), ())),
                                preferred_element_type=jnp.float32)
        cb = jnp.sum(b * b, axis=1)[None, :]
        raw = pa + cb - 2.0 * g
        tmin = jnp.min(raw, axis=1, keepdims=True)
        o = o_ref[0, 0, pl.ds(j * _BN, _BN)][None, :]
        tidx = jnp.min(jnp.where(raw == tmin, o, jnp.inf), axis=1,
                       keepdims=True)
        upd = tmin < mind
        return jnp.where(upd, tmin, mind), jnp.where(upd, tidx, mini)

    mind, mini = jax.lax.fori_loop(
        jnp.int32(0), jnp.int32(NCAND // _BN), step,
        (jnp.full((_BM, 1), jnp.inf, jnp.float32),
         jnp.zeros((_BM, 1), jnp.float32)))
    mind_ref[0] = mind
    mini_ref[0] = mini


def _finalize_body(p0_ref, p1_ref, posidx_ref, mini_ref, mind_ref,
                   posa_ref, tot_ref, pos_ref, neg_ref):
    p0 = p0_ref[...]
    p1 = p1_ref[...]

    def chunk(i, carry):
        s0, c0, s1, c1 = carry
        sl = pl.ds(i * _CH, _CH)
        u0 = posidx_ref[0, sl, :]
        v0 = mini_ref[0, sl, :]
        nhit0 = jnp.sum(((u0 == p0) & (v0 == p1)).astype(jnp.float32),
                        axis=1, keepdims=True)
        d0 = jnp.sqrt(jnp.maximum(mind_ref[0, sl, :], 1e-12))
        t0 = jnp.maximum(NEG_THRESH - d0, 0.0) ** 2
        w0 = (nhit0 == 0).astype(jnp.float32)
        u1 = mini_ref[1, sl, :]
        v1 = posidx_ref[1, sl, :]
        nhit1 = jnp.sum(((u1 == p0) & (v1 == p1)).astype(jnp.float32),
                        axis=1, keepdims=True)
        d1 = jnp.sqrt(jnp.maximum(mind_ref[1, sl, :], 1e-12))
        t1 = jnp.maximum(NEG_THRESH - d1, 0.0) ** 2
        w1 = (nhit1 == 0).astype(jnp.float32)
        return (s0 + jnp.sum(t0 * w0), c0 + jnp.sum(w0),
                s1 + jnp.sum(t1 * w1), c1 + jnp.sum(w1))

    z = jnp.float32(0.0)
    s0, c0, s1, c1 = jax.lax.fori_loop(jnp.int32(0), jnp.int32(NPOS // _CH),
                                       chunk, (z, z, z, z))

    diff = posa_ref[0] - posa_ref[1]
    nrm = jnp.sqrt(jnp.sum(diff * diff, axis=1))
    ps = jnp.sum(jnp.maximum(nrm - POS_THRESH, 0.0) ** 2) / NPOS

    nl = (s0 / jnp.maximum(c0, 1.0) + s1 / jnp.maximum(c1, 1.0)) / 2.0
    pos_ref[...] = ps.reshape(1, 1)
    neg_ref[...] = nl.reshape(1, 1)
    tot_ref[...] = (ps + nl).reshape(1, 1)


def kernel(feats0, feats1, pos_pairs):
    pp = pos_pairs.astype(jnp.int32)
    pos_idx0 = pp[_SEL, 0]
    pos_idx1 = pp[_SEL, 1]

    idx_all = jnp.concatenate([
        pos_idx0,
        pos_idx1,
        jnp.asarray(_CAND1, jnp.int32),
        jnp.asarray(_CAND0, jnp.int32),
    ]).reshape(2 * (NPOS + NCAND) // _GC, _GC)
    A, B = _sc_gather(feats0, feats1, idx_all)

    orig = jnp.stack([jnp.asarray(_CAND1, jnp.int32),
                      jnp.asarray(_CAND0, jnp.int32)]).astype(jnp.float32)
    orig = orig[:, None, :]

    grid = (2, NPOS // _BM)
    mind, mini = pl.pallas_call(
        _dist_body,
        grid=grid,
        in_specs=[
            pl.BlockSpec((1, _BM, D), lambda d, m: (d, m, _I0)),
            pl.BlockSpec((1, NCAND, D), lambda d, m: (d, _I0, _I0)),
            pl.BlockSpec((1, 1, NCAND), lambda d, m: (d, _I0, _I0)),
        ],
        out_specs=[
            pl.BlockSpec((1, _BM, 1), lambda d, m: (d, m, _I0)),
            pl.BlockSpec((1, _BM, 1), lambda d, m: (d, m, _I0)),
        ],
        out_shape=[
            jax.ShapeDtypeStruct((2, NPOS, 1), jnp.float32),
            jax.ShapeDtypeStruct((2, NPOS, 1), jnp.float32),
        ],
        interpret=_INTERPRET,
    )(A, B, orig)

    pad = jnp.full((_PPAD - NPAIRS,), -1.0, jnp.float32)
    p0f = jnp.concatenate([pp[:, 0].astype(jnp.float32), pad])[None, :]
    p1f = jnp.concatenate([pp[:, 1].astype(jnp.float32), pad])[None, :]
    posidx = jnp.stack([pos_idx0, pos_idx1]).astype(jnp.float32)[:, :, None]

    tot, posl, negl = pl.pallas_call(
        _finalize_body,
        out_shape=[
            jax.ShapeDtypeStruct((1, 1), jnp.float32),
            jax.ShapeDtypeStruct((1, 1), jnp.float32),
            jax.ShapeDtypeStruct((1, 1), jnp.float32),
        ],
        interpret=_INTERPRET,
    )(p0f, p1f, posidx, mini, mind, A)

    return tot[0, 0], posl[0, 0], negl[0, 0]
